# Initial kernel scaffold; baseline (speedup 1.0000x reference)
import jax
import jax.numpy as jnp
from jax.experimental import pallas as pl


def kernel(embeddings, anchors, positives, negatives):
    raise NotImplementedError("write your pallas kernel here")


# speedup vs baseline: 3.3035x; 3.3035x over previous
import functools

import jax
import jax.numpy as jnp
from jax import lax
from jax.experimental import pallas as pl
from jax.experimental.pallas import tpu as pltpu
from jax.experimental.pallas import tpu_sc as plsc

N_NODES = 50000
D = 128
B = 16384
K = 16

NC = 2
NS = 16
NW = NC * NS
TPW = B // NW
T = 8
NCHUNK = TPW // T

BASE_MARGIN = 1.0
MIN_MARGIN = 0.5
MAX_MARGIN = 3.0
MACL_ALPHA = 0.5
TEMPERATURE = 0.1
EPS = 1e-7
MARGIN = 1.25


def _sc_body(emb, anc, pos, neg, valp_out, valn_out,
             anc_idx, pos_idx, neg_idx,
             anc_rows, pos_rows, neg_rows,
             valp_buf, valn_buf, tbuf, pbuf, sem):
    cid = lax.axis_index("c")
    sid = lax.axis_index("s")
    wid = sid * NC + cid
    base = wid * TPW

    pltpu.sync_copy(anc.at[pl.ds(base, TPW)], anc_idx)
    pltpu.sync_copy(pos.at[pl.ds(base, TPW)], pos_idx)
    pltpu.sync_copy(neg.at[pl.ds(base * K, TPW * K)], neg_idx)

    lanes = lax.iota(jnp.int32, 16)
    sign = jnp.where(lanes == 0, -1.0, 1.0).astype(jnp.float32)

    def chunk_body(chunk, carry):
        c0 = chunk * T
        pltpu.async_copy(emb.at[anc_idx.at[pl.ds(c0, T)]], anc_rows, sem).wait()
        pltpu.async_copy(emb.at[pos_idx.at[pl.ds(c0, T)]], pos_rows, sem).wait()
        pltpu.async_copy(emb.at[neg_idx.at[pl.ds(c0 * K, T * K)]],
                         neg_rows, sem).wait()

        for t in range(T):
            av = [anc_rows[t, pl.ds(j * 16, 16)] for j in range(D // 16)]
            av0s = av[0] * sign

            for k in range(K):
                row = t * K + k
                s = av0s * neg_rows[row, pl.ds(0, 16)]
                for j in range(1, D // 16):
                    s = s + av[j] * neg_rows[row, pl.ds(j * 16, 16)]
                tbuf[k, :] = s

            acc = plsc.load_gather(tbuf, [lanes, jnp.zeros((16,), jnp.int32)])
            for c in range(1, 16):
                acc = acc + plsc.load_gather(
                    tbuf, [lanes, jnp.full((16,), c, jnp.int32)])
            lb = c0 + t
            plsc.store_scatter(valn_buf,
                               [lanes, jnp.full((16,), lb, jnp.int32)], -acc)

            s = av0s * pos_rows[t, pl.ds(0, 16)]
            for j in range(1, D // 16):
                s = s + av[j] * pos_rows[t, pl.ds(j * 16, 16)]
            pbuf[t, :] = s

        accp = plsc.load_gather(pbuf, [lanes, jnp.zeros((16,), jnp.int32)])
        for c in range(1, 16):
            accp = accp + plsc.load_gather(
                pbuf, [lanes, jnp.full((16,), c, jnp.int32)])
        idxp = jnp.minimum(c0 + lanes, TPW - 1)
        plsc.store_scatter(valp_buf, [idxp], -accp, mask=lanes < T)
        return carry

    lax.fori_loop(0, NCHUNK, chunk_body, 0)

    pltpu.sync_copy(valp_buf, valp_out.at[pl.ds(base, TPW)])
    for k in range(K):
        pltpu.sync_copy(valn_buf.at[k], valn_out.at[pl.ds(k * B + base, TPW)])


_sc_kernel = functools.partial(
    pl.kernel,
    mesh=plsc.VectorSubcoreMesh(core_axis_name="c", subcore_axis_name="s"),
    compiler_params=pltpu.CompilerParams(needs_layout_passes=False),
    out_type=[
        jax.ShapeDtypeStruct((B,), jnp.float32),
        jax.ShapeDtypeStruct((B * K,), jnp.float32),
    ],
    scratch_types=[
        pltpu.VMEM((TPW,), jnp.int32),
        pltpu.VMEM((TPW,), jnp.int32),
        pltpu.VMEM((TPW * K,), jnp.int32),
        pltpu.VMEM((T, D), jnp.float32),
        pltpu.VMEM((T, D), jnp.float32),
        pltpu.VMEM((T * K, D), jnp.float32),
        pltpu.VMEM((TPW,), jnp.float32),
        pltpu.VMEM((K, TPW), jnp.float32),
        pltpu.VMEM((16, 16), jnp.float32),
        pltpu.VMEM((16, 16), jnp.float32),
        pltpu.SemaphoreType.DMA,
    ],
)(_sc_body)


def _tc_body(vp_ref, vn_ref, loss_ref, dpos_ref, dneg_ref, acc_ref):
    vp = jnp.maximum(vp_ref[...], 1.0 + EPS)
    dp = jnp.log(vp + jnp.sqrt(vp * vp - 1.0))
    vn = jnp.maximum(vn_ref[...], 1.0 + EPS)
    dn = jnp.log(vn + jnp.sqrt(vn * vn - 1.0))
    diff = dp[None, :, :] - dn + MARGIN
    loss_ref[0, 0] = jnp.sum(jnp.maximum(diff, 0.0)) / (B * K * TEMPERATURE)
    dpos_ref[0, 0] = jnp.sum(dp) / B
    dneg_ref[0, 0] = jnp.sum(dn) / (B * K)
    acc_ref[0, 0] = jnp.sum((dp[None, :, :] < dn).astype(jnp.float32)) / (B * K)


def kernel(embeddings, anchors, positives, negatives):
    valp, valn = _sc_kernel(embeddings, anchors, positives,
                            negatives.reshape(-1))
    scalar = jax.ShapeDtypeStruct((1, 1), jnp.float32)
    outs = pl.pallas_call(
        _tc_body,
        out_shape=[scalar] * 4,
        out_specs=[pl.BlockSpec(memory_space=pltpu.SMEM)] * 4,
    )(valp.reshape(B // D, D), valn.reshape(K, B // D, D))
    loss, avg_pos, avg_neg, acc = (o[0, 0] for o in outs)
    return (loss, avg_pos, avg_neg,
            jnp.float32(MARGIN), jnp.float32(0.0), jnp.float32(0.5), acc)
